# baseline (speedup 1.0000x reference)
import functools

import jax
import jax.numpy as jnp
from jax import lax
from jax.experimental import pallas as pl
from jax.experimental.pallas import tpu as pltpu
import jax.experimental.pallas.tpu_sc as plsc

_N = 10000
_E = 320000
_R = 10240
_RPS = _R // 16
_NC = 2
_NS = 16
_NT = _NC * _NS
_CHUNK = 128
_NCH = 80
_NWAVE = 2
_NCH2 = _NCH // _NWAVE
_EPT = _CHUNK * _NCH
_EPAD = _EPT * _NT
_ZROW = _N
_JROW = _N + 8
_FB = 128
_SHIFT = 14
_MASK = (1 << _SHIFT) - 1
_BLK = 2560
_GRID = _R // _BLK


def _mm(a, w):
    return lax.dot_general(a, w, (((1,), (0,)), ((), ())),
                           precision=lax.Precision.HIGHEST,
                           preferred_element_type=jnp.float32)


def _make_lhat_sc():
    mesh = plsc.VectorSubcoreMesh(core_axis_name="c", subcore_axis_name="s",
                                  num_cores=_NC, num_subcores=_NS)

    @functools.partial(
        pl.kernel,
        out_type=jax.ShapeDtypeStruct((_NC, _R, _FB), jnp.float32),
        mesh=mesh,
        scratch_types=[
            pltpu.VMEM((_NCH2, _CHUNK), jnp.int32),
            pltpu.VMEM((_NCH2, _CHUNK), jnp.int32),
            pltpu.VMEM((_CHUNK, _FB), jnp.float32),
            pltpu.VMEM((_CHUNK, _FB), jnp.float32),
            pltpu.VMEM_SHARED((_R, _FB), jnp.float32),
            pltpu.SemaphoreType.DMA,
            pltpu.SemaphoreType.DMA,
            pltpu.SemaphoreType.DMA,
            pltpu.SemaphoreType.DMA,
        ],
    )
    def lhat(u_hbm, pk_hbm, out_hbm,
             pk_v, dst_v, buf_a, buf_b, acc, sem_a, sem_b, sem_sa, sem_sb):
        c = lax.axis_index("c")
        s = lax.axis_index("s")
        tid = c * _NS + s

        def zrow(i, carry):
            for f in range(_FB // 16):
                buf_a[i, pl.ds(f * 16, 16)] = jnp.zeros((16,), jnp.float32)
            return carry

        lax.fori_loop(0, _CHUNK, zrow, 0, unroll=False)
        for k in range(_RPS // _CHUNK):
            pltpu.sync_copy(buf_a,
                            acc.at[pl.ds(s * _RPS + k * _CHUNK, _CHUNK)])
        plsc.subcore_barrier()

        def wait_g(buf, sem):
            pltpu.make_async_copy(u_hbm.at[pk_v.at[0]], buf, sem).wait()

        def wait_s(buf, sem):
            pltpu.make_async_copy(buf, acc.at[dst_v.at[0]], sem).wait()

        for w in range(_NWAVE):
            pltpu.sync_copy(
                pk_hbm.at[pl.ds(tid * _NCH + w * _NCH2, _NCH2)], pk_v)

            def unpack(i, carry):
                for f in range(_CHUNK // 16):
                    v = pk_v[i, pl.ds(f * 16, 16)]
                    dst_v[i, pl.ds(f * 16, 16)] = jnp.bitwise_and(v, _MASK)
                    pk_v[i, pl.ds(f * 16, 16)] = jnp.right_shift(v, _SHIFT)
                return carry

            lax.fori_loop(0, _NCH2, unpack, 0, unroll=False)

            pltpu.async_copy(u_hbm.at[pk_v.at[0]], buf_a, sem_a)
            pltpu.async_copy(u_hbm.at[pk_v.at[1]], buf_b, sem_b)

            def body(it, carry):
                j0 = it * 2
                j1 = j0 + 1
                wait_g(buf_a, sem_a)
                pltpu.async_copy(buf_a, acc.at[dst_v.at[j0]], sem_sa,
                                 add=True)
                wait_g(buf_b, sem_b)
                pltpu.async_copy(buf_b, acc.at[dst_v.at[j1]], sem_sb,
                                 add=True)
                wait_s(buf_a, sem_sa)
                pltpu.async_copy(u_hbm.at[pk_v.at[j0 + 2]], buf_a, sem_a)
                wait_s(buf_b, sem_sb)
                pltpu.async_copy(u_hbm.at[pk_v.at[j1 + 2]], buf_b, sem_b)
                return carry

            lax.fori_loop(0, _NCH2 // 2 - 1, body, 0, unroll=False)

            wait_g(buf_a, sem_a)
            pltpu.async_copy(buf_a, acc.at[dst_v.at[_NCH2 - 2]], sem_sa,
                             add=True)
            wait_g(buf_b, sem_b)
            pltpu.async_copy(buf_b, acc.at[dst_v.at[_NCH2 - 1]], sem_sb,
                             add=True)
            wait_s(buf_a, sem_sa)
            wait_s(buf_b, sem_sb)
        plsc.subcore_barrier()

        pltpu.sync_copy(acc.at[pl.ds(s * _RPS, _RPS)],
                        out_hbm.at[c, pl.ds(s * _RPS, _RPS)])

    return lhat


_SC_CACHE = {}


def _lhat(u, pk2d):
    if "lhat" not in _SC_CACHE:
        _SC_CACHE["lhat"] = _make_lhat_sc()
    return _SC_CACHE["lhat"](u, pk2d)


_NSPEC = pl.BlockSpec((_BLK, _FB), lambda i: (i, 0))
_PSPEC = pl.BlockSpec((2, _BLK, _FB), lambda i: (0, i, 0))
_WSPEC = pl.BlockSpec((_FB, _FB), lambda i: (0, 0))
_BSPEC = pl.BlockSpec((1, _FB), lambda i: (0, 0))


def _node_out(k):
    return [jax.ShapeDtypeStruct((_R, _FB), jnp.float32)] * k


def _tc0_body(degp, x, dinv_o, u_o):
    deg = degp[0] + degp[1]
    rows = (pl.program_id(0) * _BLK
            + lax.broadcasted_iota(jnp.int32, deg.shape, 0))
    good = jnp.logical_and(deg > 0.5, rows < _N)
    dinv = jnp.where(good, 1.0 / jnp.sqrt(jnp.maximum(deg, 1.0)), 0.0)
    dinv_o[...] = dinv
    u_o[...] = dinv * x[...]


def _tc0(degp, x):
    return pl.pallas_call(
        _tc0_body,
        grid=(_GRID,),
        in_specs=[_PSPEC, _NSPEC],
        out_specs=[_NSPEC, _NSPEC],
        out_shape=_node_out(2),
    )(degp, x)


def _tca_body(h, wk0, ws0, bs, cheb_o, skp_o):
    cheb_o[...] = _mm(h[...], wk0[...])
    skp_o[...] = _mm(h[...], ws0[...]) + bs[...]


def _tca(h, wk0, ws0, bs):
    return pl.pallas_call(
        _tca_body,
        grid=(_GRID,),
        in_specs=[_NSPEC, _WSPEC, _WSPEC, _BSPEC],
        out_specs=[_NSPEC, _NSPEC],
        out_shape=_node_out(2),
    )(h, wk0, ws0, bs)


def _tapb_body(p, dinv, cheb, wk, tx_o, u_o, cheb_o):
    dinv_v = dinv[...]
    tx = -(dinv_v * (p[0] + p[1]))
    tx_o[...] = tx
    u_o[...] = dinv_v * tx
    cheb_o[...] = cheb[...] + _mm(tx, wk[...])


def _tapb(p, dinv, cheb, wk):
    return pl.pallas_call(
        _tapb_body,
        grid=(_GRID,),
        in_specs=[_PSPEC, _NSPEC, _NSPEC, _WSPEC],
        out_specs=[_NSPEC, _NSPEC, _NSPEC],
        out_shape=_node_out(3),
    )(p, dinv, cheb, wk)


def _tapc_body(p, dinv, tx0, cheb, wk, tx_o, u_o, cheb_o):
    dinv_v = dinv[...]
    tx = -2.0 * (dinv_v * (p[0] + p[1])) - tx0[...]
    tx_o[...] = tx
    u_o[...] = dinv_v * tx
    cheb_o[...] = cheb[...] + _mm(tx, wk[...])


def _tapc(p, dinv, tx0, cheb, wk):
    return pl.pallas_call(
        _tapc_body,
        grid=(_GRID,),
        in_specs=[_PSPEC, _NSPEC, _NSPEC, _NSPEC, _WSPEC],
        out_specs=[_NSPEC, _NSPEC, _NSPEC],
        out_shape=_node_out(3),
    )(p, dinv, tx0, cheb, wk)


def _tapd_body(p, dinv, tx1, cheb, wk, bk, skp, h_o, u_o):
    dinv_v = dinv[...]
    tx = -2.0 * (dinv_v * (p[0] + p[1])) - tx1[...]
    cheb2 = cheb[...] + _mm(tx, wk[...])
    h = jnp.maximum(cheb2 + bk[...], 0.0) + skp[...]
    h_o[...] = h
    u_o[...] = dinv_v * h


def _tapd(p, dinv, tx1, cheb, wk, bk, skp):
    return pl.pallas_call(
        _tapd_body,
        grid=(_GRID,),
        in_specs=[_PSPEC, _NSPEC, _NSPEC, _NSPEC, _WSPEC, _BSPEC, _NSPEC],
        out_specs=[_NSPEC, _NSPEC],
        out_shape=_node_out(2),
    )(p, dinv, tx1, cheb, wk, bk, skp)


def _tcmix_body(p, dinv, h, mw0, mw1, mb, out_o):
    tx1 = -(dinv[...] * (p[0] + p[1]))
    out_o[...] = _mm(h[...], mw0[...]) + _mm(tx1, mw1[...]) + mb[...]


def _tcmix(p, dinv, h, mw0, mw1, mb):
    return pl.pallas_call(
        _tcmix_body,
        grid=(_GRID,),
        in_specs=[_PSPEC, _NSPEC, _NSPEC, _WSPEC, _WSPEC, _BSPEC],
        out_specs=_NSPEC,
        out_shape=_node_out(1)[0],
    )(p, dinv, h, mw0, mw1, mb)


def _padw(w):
    k, fi, fo = w.shape
    return jnp.zeros((k, _FB, _FB), jnp.float32).at[:, :fi, :fo].set(w)


def _padb(b):
    return jnp.zeros((1, _FB), jnp.float32).at[0, :b.shape[0]].set(b)


def kernel(x, edge_index,
           kipf_W0, kipf_b0, kipf_W1, kipf_b1, kipf_W2, kipf_b2,
           skip_W0, skip_b0, skip_W1, skip_b1, skip_W2, skip_b2,
           mix_W, mix_b):
    src = edge_index[0].astype(jnp.int32)
    dst = edge_index[1].astype(jnp.int32)
    srcp = jnp.where(src == dst, _ZROW, src)
    npad = _EPAD - _E
    padpk = jnp.full((npad,), (_ZROW << _SHIFT) | _JROW, jnp.int32)
    pk = jnp.concatenate([(srcp << _SHIFT) | dst, padpk])
    pk2d = pk.reshape(_NT * _NCH, _CHUNK)

    pkd = jnp.concatenate([(srcp << _SHIFT) | src, padpk])
    pkd2d = pkd.reshape(_NT * _NCH, _CHUNK)
    ones_tab = jnp.zeros((_R, _FB), jnp.float32).at[:_N].set(1.0)

    xp = jnp.zeros((_R, _FB), jnp.float32).at[:_N].set(x)

    kipf = [(_padw(kipf_W0), _padb(kipf_b0)),
            (_padw(kipf_W1), _padb(kipf_b1)),
            (_padw(kipf_W2), _padb(kipf_b2))]
    skip = [(_padw(skip_W0), _padb(skip_b0)),
            (_padw(skip_W1), _padb(skip_b1)),
            (_padw(skip_W2), _padb(skip_b2))]
    mixw = _padw(mix_W)
    mixb = _padb(mix_b)

    degp = _lhat(ones_tab, pkd2d)
    dinv, u = _tc0(degp, xp)

    h = xp
    for l in range(3):
        Wk, bk = kipf[l]
        Ws, bs = skip[l]
        cheb, skp = _tca(h, Wk[0], Ws[0], bs)
        p = _lhat(u, pk2d)
        tx1, u, cheb = _tapb(p, dinv, cheb, Wk[1])
        p = _lhat(u, pk2d)
        tx2, u, cheb = _tapc(p, dinv, h, cheb, Wk[2])
        p = _lhat(u, pk2d)
        h, u = _tapd(p, dinv, tx1, cheb, Wk[3], bk, skp)

    p = _lhat(u, pk2d)
    out = _tcmix(p, dinv, h, mixw[0], mixw[1], mixb)
    return out[:_N, :64]


# speedup vs baseline: 1.0100x; 1.0100x over previous
import functools

import jax
import jax.numpy as jnp
from jax import lax
from jax.experimental import pallas as pl
from jax.experimental.pallas import tpu as pltpu
import jax.experimental.pallas.tpu_sc as plsc

_N = 10000
_E = 320000
_R = 10240
_RPS = _R // 16
_NC = 2
_NS = 16
_NT = _NC * _NS
_CHUNK = 128
_NCH = 80
_NWAVE = 2
_NCH2 = _NCH // _NWAVE
_EPT = _CHUNK * _NCH
_EPAD = _EPT * _NT
_ZROW = _N
_JROW = _N + 8
_FB = 128
_SHIFT = 14
_MASK = (1 << _SHIFT) - 1
_BLK = 2560
_GRID = _R // _BLK


def _mm(a, w):
    return lax.dot_general(a, w, (((1,), (0,)), ((), ())),
                           precision=lax.Precision.HIGHEST,
                           preferred_element_type=jnp.float32)


def _make_lhat_sc():
    mesh = plsc.VectorSubcoreMesh(core_axis_name="c", subcore_axis_name="s",
                                  num_cores=_NC, num_subcores=_NS)

    @functools.partial(
        pl.kernel,
        out_type=jax.ShapeDtypeStruct((_NC, _R, _FB), jnp.float32),
        mesh=mesh,
        scratch_types=[
            pltpu.VMEM((_NCH2, _CHUNK), jnp.int32),
            pltpu.VMEM((_NCH2, _CHUNK), jnp.int32),
            pltpu.VMEM((_CHUNK, _FB), jnp.float32),
            pltpu.VMEM((_CHUNK, _FB), jnp.float32),
            pltpu.VMEM_SHARED((_R, _FB), jnp.float32),
            pltpu.SemaphoreType.DMA,
            pltpu.SemaphoreType.DMA,
            pltpu.SemaphoreType.DMA,
            pltpu.SemaphoreType.DMA,
        ],
    )
    def lhat(u_hbm, pk_hbm, out_hbm,
             pk_v, dst_v, buf_a, buf_b, acc, sem_a, sem_b, sem_sa, sem_sb):
        c = lax.axis_index("c")
        s = lax.axis_index("s")
        tid = c * _NS + s

        def zrow(i, carry):
            for f in range(_FB // 16):
                buf_a[i, pl.ds(f * 16, 16)] = jnp.zeros((16,), jnp.float32)
            return carry

        lax.fori_loop(0, _CHUNK, zrow, 0, unroll=False)
        for k in range(_RPS // _CHUNK):
            pltpu.sync_copy(buf_a,
                            acc.at[pl.ds(s * _RPS + k * _CHUNK, _CHUNK)])
        plsc.subcore_barrier()

        def wait_g(buf, sem):
            pltpu.make_async_copy(u_hbm.at[pk_v.at[0]], buf, sem).wait()

        def wait_s(buf, sem):
            pltpu.make_async_copy(buf, acc.at[dst_v.at[0]], sem).wait()

        for w in range(_NWAVE):
            pltpu.sync_copy(
                pk_hbm.at[pl.ds(tid * _NCH + w * _NCH2, _NCH2)], pk_v)

            def unpack(i, carry):
                for f in range(_CHUNK // 16):
                    v = pk_v[i, pl.ds(f * 16, 16)]
                    dst_v[i, pl.ds(f * 16, 16)] = jnp.bitwise_and(v, _MASK)
                    pk_v[i, pl.ds(f * 16, 16)] = jnp.right_shift(v, _SHIFT)
                return carry

            lax.fori_loop(0, _NCH2, unpack, 0, unroll=False)

            pltpu.async_copy(u_hbm.at[pk_v.at[0]], buf_a, sem_a)
            pltpu.async_copy(u_hbm.at[pk_v.at[1]], buf_b, sem_b)

            def body(it, carry):
                j0 = it * 2
                j1 = j0 + 1
                wait_g(buf_a, sem_a)
                pltpu.async_copy(buf_a, acc.at[dst_v.at[j0]], sem_sa,
                                 add=True)
                wait_g(buf_b, sem_b)
                wait_s(buf_a, sem_sa)
                pltpu.async_copy(u_hbm.at[pk_v.at[j0 + 2]], buf_a, sem_a)
                pltpu.async_copy(buf_b, acc.at[dst_v.at[j1]], sem_sb,
                                 add=True)
                wait_s(buf_b, sem_sb)
                pltpu.async_copy(u_hbm.at[pk_v.at[j1 + 2]], buf_b, sem_b)
                return carry

            lax.fori_loop(0, _NCH2 // 2 - 1, body, 0, unroll=False)

            wait_g(buf_a, sem_a)
            pltpu.async_copy(buf_a, acc.at[dst_v.at[_NCH2 - 2]], sem_sa,
                             add=True)
            wait_g(buf_b, sem_b)
            wait_s(buf_a, sem_sa)
            pltpu.async_copy(buf_b, acc.at[dst_v.at[_NCH2 - 1]], sem_sb,
                             add=True)
            wait_s(buf_b, sem_sb)
        plsc.subcore_barrier()

        pltpu.sync_copy(acc.at[pl.ds(s * _RPS, _RPS)],
                        out_hbm.at[c, pl.ds(s * _RPS, _RPS)])

    return lhat


_SC_CACHE = {}


def _lhat(u, pk2d):
    if "lhat" not in _SC_CACHE:
        _SC_CACHE["lhat"] = _make_lhat_sc()
    return _SC_CACHE["lhat"](u, pk2d)


_NSPEC = pl.BlockSpec((_BLK, _FB), lambda i: (i, 0))
_PSPEC = pl.BlockSpec((2, _BLK, _FB), lambda i: (0, i, 0))
_WSPEC = pl.BlockSpec((_FB, _FB), lambda i: (0, 0))
_BSPEC = pl.BlockSpec((1, _FB), lambda i: (0, 0))


def _node_out(k):
    return [jax.ShapeDtypeStruct((_R, _FB), jnp.float32)] * k


def _tc0_body(degp, x, dinv_o, u_o):
    deg = degp[0] + degp[1]
    rows = (pl.program_id(0) * _BLK
            + lax.broadcasted_iota(jnp.int32, deg.shape, 0))
    good = jnp.logical_and(deg > 0.5, rows < _N)
    dinv = jnp.where(good, 1.0 / jnp.sqrt(jnp.maximum(deg, 1.0)), 0.0)
    dinv_o[...] = dinv
    u_o[...] = dinv * x[...]


def _tc0(degp, x):
    return pl.pallas_call(
        _tc0_body,
        grid=(_GRID,),
        in_specs=[_PSPEC, _NSPEC],
        out_specs=[_NSPEC, _NSPEC],
        out_shape=_node_out(2),
    )(degp, x)


def _tca_body(h, wk0, ws0, bs, cheb_o, skp_o):
    cheb_o[...] = _mm(h[...], wk0[...])
    skp_o[...] = _mm(h[...], ws0[...]) + bs[...]


def _tca(h, wk0, ws0, bs):
    return pl.pallas_call(
        _tca_body,
        grid=(_GRID,),
        in_specs=[_NSPEC, _WSPEC, _WSPEC, _BSPEC],
        out_specs=[_NSPEC, _NSPEC],
        out_shape=_node_out(2),
    )(h, wk0, ws0, bs)


def _tapb_body(p, dinv, cheb, wk, tx_o, u_o, cheb_o):
    dinv_v = dinv[...]
    tx = -(dinv_v * (p[0] + p[1]))
    tx_o[...] = tx
    u_o[...] = dinv_v * tx
    cheb_o[...] = cheb[...] + _mm(tx, wk[...])


def _tapb(p, dinv, cheb, wk):
    return pl.pallas_call(
        _tapb_body,
        grid=(_GRID,),
        in_specs=[_PSPEC, _NSPEC, _NSPEC, _WSPEC],
        out_specs=[_NSPEC, _NSPEC, _NSPEC],
        out_shape=_node_out(3),
    )(p, dinv, cheb, wk)


def _tapc_body(p, dinv, tx0, cheb, wk, tx_o, u_o, cheb_o):
    dinv_v = dinv[...]
    tx = -2.0 * (dinv_v * (p[0] + p[1])) - tx0[...]
    tx_o[...] = tx
    u_o[...] = dinv_v * tx
    cheb_o[...] = cheb[...] + _mm(tx, wk[...])


def _tapc(p, dinv, tx0, cheb, wk):
    return pl.pallas_call(
        _tapc_body,
        grid=(_GRID,),
        in_specs=[_PSPEC, _NSPEC, _NSPEC, _NSPEC, _WSPEC],
        out_specs=[_NSPEC, _NSPEC, _NSPEC],
        out_shape=_node_out(3),
    )(p, dinv, tx0, cheb, wk)


def _tapd_body(p, dinv, tx1, cheb, wk, bk, skp, h_o, u_o):
    dinv_v = dinv[...]
    tx = -2.0 * (dinv_v * (p[0] + p[1])) - tx1[...]
    cheb2 = cheb[...] + _mm(tx, wk[...])
    h = jnp.maximum(cheb2 + bk[...], 0.0) + skp[...]
    h_o[...] = h
    u_o[...] = dinv_v * h


def _tapd(p, dinv, tx1, cheb, wk, bk, skp):
    return pl.pallas_call(
        _tapd_body,
        grid=(_GRID,),
        in_specs=[_PSPEC, _NSPEC, _NSPEC, _NSPEC, _WSPEC, _BSPEC, _NSPEC],
        out_specs=[_NSPEC, _NSPEC],
        out_shape=_node_out(2),
    )(p, dinv, tx1, cheb, wk, bk, skp)


def _tcmix_body(p, dinv, h, mw0, mw1, mb, out_o):
    tx1 = -(dinv[...] * (p[0] + p[1]))
    out_o[...] = _mm(h[...], mw0[...]) + _mm(tx1, mw1[...]) + mb[...]


def _tcmix(p, dinv, h, mw0, mw1, mb):
    return pl.pallas_call(
        _tcmix_body,
        grid=(_GRID,),
        in_specs=[_PSPEC, _NSPEC, _NSPEC, _WSPEC, _WSPEC, _BSPEC],
        out_specs=_NSPEC,
        out_shape=_node_out(1)[0],
    )(p, dinv, h, mw0, mw1, mb)


def _padw(w):
    k, fi, fo = w.shape
    return jnp.zeros((k, _FB, _FB), jnp.float32).at[:, :fi, :fo].set(w)


def _padb(b):
    return jnp.zeros((1, _FB), jnp.float32).at[0, :b.shape[0]].set(b)


def kernel(x, edge_index,
           kipf_W0, kipf_b0, kipf_W1, kipf_b1, kipf_W2, kipf_b2,
           skip_W0, skip_b0, skip_W1, skip_b1, skip_W2, skip_b2,
           mix_W, mix_b):
    src = edge_index[0].astype(jnp.int32)
    dst = edge_index[1].astype(jnp.int32)
    srcp = jnp.where(src == dst, _ZROW, src)
    npad = _EPAD - _E
    padpk = jnp.full((npad,), (_ZROW << _SHIFT) | _JROW, jnp.int32)
    pk = jnp.concatenate([(srcp << _SHIFT) | dst, padpk])
    pk2d = pk.reshape(_NT * _NCH, _CHUNK)

    pkd = jnp.concatenate([(srcp << _SHIFT) | src, padpk])
    pkd2d = pkd.reshape(_NT * _NCH, _CHUNK)
    ones_tab = jnp.zeros((_R, _FB), jnp.float32).at[:_N].set(1.0)

    xp = jnp.zeros((_R, _FB), jnp.float32).at[:_N].set(x)

    kipf = [(_padw(kipf_W0), _padb(kipf_b0)),
            (_padw(kipf_W1), _padb(kipf_b1)),
            (_padw(kipf_W2), _padb(kipf_b2))]
    skip = [(_padw(skip_W0), _padb(skip_b0)),
            (_padw(skip_W1), _padb(skip_b1)),
            (_padw(skip_W2), _padb(skip_b2))]
    mixw = _padw(mix_W)
    mixb = _padb(mix_b)

    degp = _lhat(ones_tab, pkd2d)
    dinv, u = _tc0(degp, xp)

    h = xp
    for l in range(3):
        Wk, bk = kipf[l]
        Ws, bs = skip[l]
        cheb, skp = _tca(h, Wk[0], Ws[0], bs)
        p = _lhat(u, pk2d)
        tx1, u, cheb = _tapb(p, dinv, cheb, Wk[1])
        p = _lhat(u, pk2d)
        tx2, u, cheb = _tapc(p, dinv, h, cheb, Wk[2])
        p = _lhat(u, pk2d)
        h, u = _tapd(p, dinv, tx1, cheb, Wk[3], bk, skp)

    p = _lhat(u, pk2d)
    out = _tcmix(p, dinv, h, mixw[0], mixw[1], mixb)
    return out[:_N, :64]
